# Initial kernel scaffold; baseline (speedup 1.0000x reference)
import jax
import jax.numpy as jnp
from jax.experimental import pallas as pl


def kernel(idx, emb):
    raise NotImplementedError("write your pallas kernel here")


# speedup vs baseline: 1.5754x; 1.5754x over previous
import functools

import jax
import jax.numpy as jnp
from jax import lax
from jax.experimental import pallas as pl
from jax.experimental.pallas import tpu as pltpu
from jax.experimental.pallas import tpu_sc as plsc

NC = 2
NS = 16
NW = NC * NS
CH = 128
NBUF = 4


@functools.partial(jax.jit, static_argnames=("n_chunks",))
def _gather_flat(emb, idx3, n_chunks):
    V, D = emb.shape
    per_w = n_chunks * CH
    N = NW * per_w
    mesh = plsc.VectorSubcoreMesh(
        core_axis_name="c", subcore_axis_name="s", num_cores=NC, num_subcores=NS
    )

    @functools.partial(
        pl.kernel,
        out_type=jax.ShapeDtypeStruct((N, D), jnp.float32),
        mesh=mesh,
        compiler_params=pltpu.CompilerParams(use_tc_tiling_on_sc=False),
        scratch_types=[
            pltpu.VMEM((n_chunks, CH), jnp.int32),
            pltpu.VMEM((NBUF, CH, D), jnp.float32),
        ]
        + [pltpu.SemaphoreType.DMA] * NBUF,
    )
    def gather_k(emb_hbm, idx_hbm, out_hbm, idx_v, rows_v, *gsems):
        wid = lax.axis_index("s") * NC + lax.axis_index("c")
        base = wid * per_w
        pltpu.sync_copy(idx_hbm.at[wid], idx_v)

        def start(j, b):
            pltpu.async_copy(emb_hbm.at[idx_v.at[j]], rows_v.at[b], gsems[b])

        def finish(j, b):
            pltpu.make_async_copy(
                emb_hbm.at[idx_v.at[j]], rows_v.at[b], gsems[b]
            ).wait()
            pltpu.sync_copy(rows_v.at[b], out_hbm.at[pl.ds(base + j * CH, CH)])

        for b in range(NBUF):
            start(b, b)

        groups = n_chunks // NBUF

        def body(g, carry):
            for b in range(NBUF):
                j = g * NBUF + b
                finish(j, b)
                start(j + NBUF, b)
            return carry

        lax.fori_loop(0, groups - 1, body, 0, unroll=False)

        for b in range(NBUF):
            j = (groups - 1) * NBUF + b
            finish(j, b)

    return gather_k(emb, idx3)


def kernel(idx, emb):
    B, F = idx.shape
    V, D = emb.shape
    N = B * F
    per_w = N // NW
    n_chunks = per_w // CH
    assert per_w % CH == 0 and N % NW == 0
    idx3 = idx.reshape(NW, n_chunks, CH).astype(jnp.int32)
    out = _gather_flat(emb, idx3, n_chunks)
    return out.reshape(B, F, D)
